# baseline (speedup 1.0000x reference)
import functools

import jax
import jax.numpy as jnp
from jax import lax
from jax.experimental import pallas as pl
from jax.experimental.pallas import tpu as pltpu
from jax.experimental.pallas import tpu_sc as plsc

_NC = 2
_NS = 16
_NW = _NC * _NS


def _sc_gather(weight, indices):
    k_short = indices.shape[0]
    d = weight.shape[1]
    rows_per_w = k_short // _NW
    chunk = min(rows_per_w, 128)
    mesh = plsc.VectorSubcoreMesh(core_axis_name="c", subcore_axis_name="s")

    @functools.partial(
        pl.kernel,
        mesh=mesh,
        out_type=jax.ShapeDtypeStruct((k_short, d), jnp.float32),
        scratch_types=[
            pltpu.VMEM((rows_per_w,), jnp.int32),
            pltpu.VMEM((chunk, d), jnp.float32),
            pltpu.VMEM((chunk, d), jnp.float32),
            pltpu.SemaphoreType.DMA,
            pltpu.SemaphoreType.DMA,
            pltpu.SemaphoreType.DMA,
            pltpu.SemaphoreType.DMA,
        ],
    )
    def gather_kernel(w_hbm, idx_hbm, w_out, idx_v, rows_v0, rows_v1,
                      gsem0, gsem1, osem0, osem1):
        rows_b = (rows_v0, rows_v1)
        gsem = (gsem0, gsem1)
        osem = (osem0, osem1)
        wid = lax.axis_index("s") * _NC + lax.axis_index("c")
        base = wid * rows_per_w
        pltpu.sync_copy(idx_hbm.at[pl.ds(base, rows_per_w)], idx_v)

        n_chunks = rows_per_w // chunk

        def fire_gather(c, b):
            return pltpu.async_copy(
                w_hbm.at[idx_v.at[pl.ds(c * chunk, chunk)]], rows_b[b],
                gsem[b])

        def fire_out(c, b):
            return pltpu.async_copy(
                rows_b[b], w_out.at[pl.ds(base + c * chunk, chunk)], osem[b])

        pending_g = {0: fire_gather(0, 0)}
        pending_o = {}
        for c in range(n_chunks):
            b = c & 1
            if c + 1 < n_chunks:
                if c >= 1:
                    pending_o.pop(c - 1).wait()
                pending_g[c + 1] = fire_gather(c + 1, 1 - b)
            pending_g.pop(c).wait()
            pending_o[c] = fire_out(c, b)
        for c in sorted(pending_o):
            pending_o.pop(c).wait()

    return gather_kernel(weight, indices)


def _tc_body(n_j, d, x_ref, w_ref, at_ref, b_ref, o_ref):
    att = at_ref[...]
    m = jnp.max(att, axis=1, keepdims=True)
    e = jnp.exp(att - m)
    a = e / jnp.sum(e, axis=1, keepdims=True)
    w_f = w_ref[...]
    acc = jnp.broadcast_to(b_ref[...], o_ref.shape)
    for j in range(n_j):
        wj = (w_f * a[:, j:j + 1]).astype(jnp.bfloat16)
        xj = x_ref[:, j * d:(j + 1) * d]
        acc = acc + lax.dot_general(xj, wj, (((1,), (1,)), ((), ())),
                                    preferred_element_type=jnp.float32)
    o_ref[...] = acc


def _tc_matmul(x, w_g, att_g, bias_r, kblk=2048):
    bsz, three_i = x.shape
    k_short, d = w_g.shape
    n_j = three_i // d

    return pl.pallas_call(
        functools.partial(_tc_body, n_j, d),
        grid=(k_short // kblk,),
        in_specs=[
            pl.BlockSpec((bsz, three_i), lambda i: (0, 0)),
            pl.BlockSpec((kblk, d), lambda i: (i, 0)),
            pl.BlockSpec((kblk, n_j), lambda i: (i, 0)),
            pl.BlockSpec((1, kblk), lambda i: (0, i)),
        ],
        out_specs=pl.BlockSpec((bsz, kblk), lambda i: (0, i)),
        out_shape=jax.ShapeDtypeStruct((bsz, k_short), jnp.float32),
    )(x, w_g, att_g, bias_r)


def kernel(x, indices, weight, bias, attention_weights):
    k_short = indices.shape[0]
    w_g = _sc_gather(weight, indices)
    idx_dep = indices + (w_g[0, 0] * 0.0).astype(jnp.int32)
    att_g = attention_weights.at[idx_dep].get(
        mode="promise_in_bounds")
    bias_r = bias.at[idx_dep].get(
        mode="promise_in_bounds").reshape(1, k_short)
    return _tc_matmul(x.astype(jnp.bfloat16), w_g, att_g, bias_r)


# speedup vs baseline: 1.0028x; 1.0028x over previous
import functools

import jax
import jax.numpy as jnp
from jax import lax
from jax.experimental import pallas as pl
from jax.experimental.pallas import tpu as pltpu
from jax.experimental.pallas import tpu_sc as plsc

_NC = 2
_NS = 16
_NW = _NC * _NS


def _sc_gather(weight, indices):
    k_short = indices.shape[0]
    d = weight.shape[1]
    rows_per_w = k_short // _NW
    chunk = min(rows_per_w, 128)
    mesh = plsc.VectorSubcoreMesh(core_axis_name="c", subcore_axis_name="s")

    @functools.partial(
        pl.kernel,
        mesh=mesh,
        out_type=jax.ShapeDtypeStruct((k_short, d), jnp.float32),
        scratch_types=[
            pltpu.VMEM((rows_per_w,), jnp.int32),
            pltpu.VMEM((chunk, d), jnp.float32),
            pltpu.VMEM((chunk, d), jnp.float32),
            pltpu.SemaphoreType.DMA,
            pltpu.SemaphoreType.DMA,
            pltpu.SemaphoreType.DMA,
            pltpu.SemaphoreType.DMA,
        ],
    )
    def gather_kernel(w_hbm, idx_hbm, w_out, idx_v, rows_v0, rows_v1,
                      gsem0, gsem1, osem0, osem1):
        rows_b = (rows_v0, rows_v1)
        gsem = (gsem0, gsem1)
        osem = (osem0, osem1)
        wid = lax.axis_index("s") * _NC + lax.axis_index("c")
        base = wid * rows_per_w
        pltpu.sync_copy(idx_hbm.at[pl.ds(base, rows_per_w)], idx_v)

        n_chunks = rows_per_w // chunk

        def fire_gather(c, b):
            return pltpu.async_copy(
                w_hbm.at[idx_v.at[pl.ds(c * chunk, chunk)]], rows_b[b],
                gsem[b])

        def fire_out(c, b):
            return pltpu.async_copy(
                rows_b[b], w_out.at[pl.ds(base + c * chunk, chunk)], osem[b])

        pending_g = {0: fire_gather(0, 0)}
        pending_o = {}
        for c in range(n_chunks):
            b = c & 1
            if c + 1 < n_chunks:
                if c >= 1:
                    pending_o.pop(c - 1).wait()
                pending_g[c + 1] = fire_gather(c + 1, 1 - b)
            pending_g.pop(c).wait()
            pending_o[c] = fire_out(c, b)
        for c in sorted(pending_o):
            pending_o.pop(c).wait()

    return gather_kernel(weight, indices)


def _tc_body(n_j, d, x_ref, w_ref, at_ref, b_ref, o_ref):
    att = at_ref[...]
    m = jnp.max(att, axis=1, keepdims=True)
    e = jnp.exp(att - m)
    a = e / jnp.sum(e, axis=1, keepdims=True)
    w_f = w_ref[...]
    acc = jnp.broadcast_to(b_ref[...], o_ref.shape)
    for j in range(n_j):
        wj = (w_f * a[:, j:j + 1]).astype(jnp.bfloat16)
        xj = x_ref[:, j * d:(j + 1) * d]
        acc = acc + lax.dot_general(xj, wj, (((1,), (1,)), ((), ())),
                                    preferred_element_type=jnp.float32)
    o_ref[...] = acc


def _tc_matmul(x, w_g, att_g, bias_r, kblk=2048):
    bsz, three_i = x.shape
    k_short, d = w_g.shape
    n_j = three_i // d

    return pl.pallas_call(
        functools.partial(_tc_body, n_j, d),
        grid=(k_short // kblk,),
        in_specs=[
            pl.BlockSpec((bsz, three_i), lambda i: (0, 0)),
            pl.BlockSpec((kblk, d), lambda i: (i, 0)),
            pl.BlockSpec((kblk, n_j), lambda i: (i, 0)),
            pl.BlockSpec((1, kblk), lambda i: (0, i)),
        ],
        out_specs=pl.BlockSpec((bsz, kblk), lambda i: (0, i)),
        out_shape=jax.ShapeDtypeStruct((bsz, k_short), jnp.float32),
    )(x, w_g, att_g, bias_r)


def kernel(x, indices, weight, bias, attention_weights):
    k_short = indices.shape[0]
    w_g = _sc_gather(weight, indices)
    idx_dep = indices + (w_g[0, 0] * 0.0).astype(jnp.int32)
    att_g = attention_weights.at[idx_dep].get(
        mode="promise_in_bounds")
    bias_r = bias.at[indices].get(
        mode="promise_in_bounds").reshape(1, k_short)
    return _tc_matmul(x.astype(jnp.bfloat16), w_g, att_g, bias_r)
